# baseline (speedup 1.0000x reference)
import functools

import jax
import jax.numpy as jnp
from jax import lax
from jax.experimental import pallas as pl
from jax.experimental.pallas import tpu as pltpu
from jax.experimental.pallas import tpu_sc as plsc

D = 128
NW = 32
NBUF = 4
NSPLIT = 4

_mesh = plsc.VectorSubcoreMesh(
    core_axis_name="c", subcore_axis_name="s", num_cores=2, num_subcores=16
)


@functools.partial(jax.jit, static_argnames=("nb", "s"))
def _gather_part(table, idx_part, nb, s):
    nch = nb // NW
    ng = nch // NBUF

    @functools.partial(
        pl.kernel,
        mesh=_mesh,
        out_type=jax.ShapeDtypeStruct((nb, s, D), jnp.float32),
        scratch_types=[
            pltpu.VMEM((nch, s), jnp.int32),
            pltpu.VMEM((NBUF, s, D), jnp.float32),
            pltpu.SemaphoreType.DMA((NBUF,)),
            pltpu.SemaphoreType.DMA((NBUF,)),
        ],
    )
    def body(table_hbm, idx_hbm, out_hbm, idx_v, rows_v, gsem, wsem):
        wid = lax.axis_index("s") * 2 + lax.axis_index("c")
        base = wid * nch

        def wait_gather(bf):
            pltpu.make_async_copy(
                table_hbm.at[idx_v.at[0]], rows_v.at[bf], gsem.at[bf]
            ).wait()

        def wait_write(bf):
            pltpu.make_async_copy(
                rows_v.at[bf], out_hbm.at[0], wsem.at[bf]
            ).wait()

        pltpu.sync_copy(idx_hbm.at[wid], idx_v)

        for bf in range(NBUF):
            pltpu.async_copy(
                table_hbm.at[idx_v.at[bf]], rows_v.at[bf], gsem.at[bf]
            )

        def group_body(g, _):
            for bf in range(NBUF):
                j = g * NBUF + bf
                wait_gather(bf)
                pltpu.async_copy(
                    rows_v.at[bf], out_hbm.at[base + j], wsem.at[bf]
                )
            for bf in range(NBUF):
                jn = (g + 1) * NBUF + bf
                wait_write(bf)
                pltpu.async_copy(
                    table_hbm.at[idx_v.at[jn]], rows_v.at[bf], gsem.at[bf]
                )
            return 0

        lax.fori_loop(0, ng - 1, group_body, 0)

        for bf in range(NBUF):
            j = (ng - 1) * NBUF + bf
            wait_gather(bf)
            pltpu.async_copy(
                rows_v.at[bf], out_hbm.at[base + j], wsem.at[bf]
            )
        for bf in range(NBUF):
            wait_write(bf)

    return body(table, idx_part)


def _lookup(table, idx, b, s):
    nb = b // NSPLIT
    parts = []
    for k in range(NSPLIT):
        part_idx = idx[k * nb:(k + 1) * nb].reshape(NW, nb // NW, s)
        parts.append(_gather_part(table, part_idx, nb, s))
    return jnp.concatenate(parts, axis=0)


def kernel(input, support, table):
    b, s = input.shape
    inp = input.astype(jnp.int32)
    sup = support.astype(jnp.int32)
    return (_lookup(table, inp, b, s), _lookup(table, sup, b, s))


# speedup vs baseline: 3.0073x; 3.0073x over previous
import functools

import jax
import jax.numpy as jnp
from jax import lax
from jax.experimental import pallas as pl
from jax.experimental.pallas import tpu as pltpu
from jax.experimental.pallas import tpu_sc as plsc

D = 128
NW = 32
CHUNK = 128
NBUF = 5

_mesh = plsc.VectorSubcoreMesh(
    core_axis_name="c", subcore_axis_name="s", num_cores=2, num_subcores=16
)


@functools.partial(jax.jit, static_argnames=("n", "nch"))
def _dual_gather(table, inp_idx, sup_idx, n, nch):
    ng = nch // NBUF

    @functools.partial(
        pl.kernel,
        mesh=_mesh,
        out_type=[
            jax.ShapeDtypeStruct((n, D), jnp.float32),
            jax.ShapeDtypeStruct((n, D), jnp.float32),
        ],
        scratch_types=[
            pltpu.VMEM((nch, CHUNK), jnp.int32),
            pltpu.VMEM((NBUF, CHUNK, D), jnp.float32),
            pltpu.SemaphoreType.DMA((NBUF,)),
            pltpu.SemaphoreType.DMA((NBUF,)),
        ],
    )
    def body(table_hbm, inp_hbm, sup_hbm, out1_hbm, out2_hbm,
             idx_v, rows_v, gsem, wsem):
        wid = lax.axis_index("s") * 2 + lax.axis_index("c")
        per_w = nch * CHUNK
        base = wid * per_w

        def wait_gather(bf):
            pltpu.make_async_copy(
                table_hbm.at[idx_v.at[0]], rows_v.at[bf], gsem.at[bf]
            ).wait()

        def wait_write(bf, out_hbm):
            pltpu.make_async_copy(
                rows_v.at[bf], out_hbm.at[pl.ds(base, CHUNK)], wsem.at[bf]
            ).wait()

        def one_lookup(idx_hbm, out_hbm):
            pltpu.sync_copy(idx_hbm.at[wid], idx_v)

            for bf in range(NBUF):
                pltpu.async_copy(
                    table_hbm.at[idx_v.at[bf]], rows_v.at[bf], gsem.at[bf]
                )

            def group_body(g, _):
                for bf in range(NBUF):
                    j = g * NBUF + bf
                    wait_gather(bf)
                    pltpu.async_copy(
                        rows_v.at[bf],
                        out_hbm.at[pl.ds(base + j * CHUNK, CHUNK)],
                        wsem.at[bf],
                    )
                for bf in range(NBUF):
                    jn = (g + 1) * NBUF + bf
                    wait_write(bf, out_hbm)
                    pltpu.async_copy(
                        table_hbm.at[idx_v.at[jn]], rows_v.at[bf], gsem.at[bf]
                    )
                return 0

            lax.fori_loop(0, ng - 1, group_body, 0)

            for bf in range(NBUF):
                j = (ng - 1) * NBUF + bf
                wait_gather(bf)
                pltpu.async_copy(
                    rows_v.at[bf],
                    out_hbm.at[pl.ds(base + j * CHUNK, CHUNK)],
                    wsem.at[bf],
                )
            for bf in range(NBUF):
                wait_write(bf, out_hbm)

        one_lookup(inp_hbm, out1_hbm)
        one_lookup(sup_hbm, out2_hbm)

    return body(table, inp_idx, sup_idx)


def kernel(input, support, table):
    b, s = input.shape
    n = b * s
    nch = n // (NW * CHUNK)
    inp = input.T.reshape(NW, nch, CHUNK).astype(jnp.int32)
    sup = support.T.reshape(NW, nch, CHUNK).astype(jnp.int32)
    out1, out2 = _dual_gather(table, inp, sup, n, nch)
    out1 = out1.reshape(s, b, D).transpose(1, 0, 2)
    out2 = out2.reshape(s, b, D).transpose(1, 0, 2)
    return (out1, out2)
